# baseline (speedup 1.0000x reference)
import jax
import jax.numpy as jnp
from jax import lax
from jax.experimental import pallas as pl
from jax.experimental.pallas import tpu as pltpu

N_ROWS = 800000
N_AE = 32
N_DE = 16
N_SEG_ATOM = 50000

B = 1280
NB = N_ROWS // B
W = 128
TR = 50432
TE = 128
FS = 40

_C00 = (((0,), (0,)), ((), ()))
_BF = jnp.bfloat16
_F32 = jnp.float32


def _split_hi_lo(v):
    hi = v.astype(_BF)
    lo = (v - hi.astype(_F32)).astype(_BF)
    return hi, lo


def _k1a_body(lo_ref, hi_ref, x_ref, aidx_ref, eidx_ref, aacc_ref, eacc_ref):
    i = pl.program_id(0)

    @pl.when(i == 0)
    def _():
        aacc_ref[...] = jnp.zeros((TR, FS), _F32)
        eacc_ref[...] = jnp.zeros((TE, FS), _F32)

    aidx_row = aidx_ref[0]
    eidx_row = eidx_ref[0]

    lane40 = lax.broadcasted_iota(jnp.int32, (B, FS), 1)
    x40a = jnp.where(lane40 < N_AE, x_ref[:, 0:FS], 1.0)
    x40e = jnp.where(lane40 >= 8, x_ref[:, 24:64], 1.0)
    xa_hi, xa_lo = _split_hi_lo(x40a)
    xe_hi, xe_lo = _split_hi_lo(x40e)

    sub_e = lax.broadcasted_iota(jnp.int32, (TE, B), 0)
    ohe = (sub_e == eidx_row).astype(_BF)
    eacc_ref[...] += (jnp.dot(ohe, xe_hi, preferred_element_type=_F32)
                      + jnp.dot(ohe, xe_lo, preferred_element_type=_F32))

    lo = lo_ref[i]
    hi = hi_ref[i]
    base = (lo // 8) * 8
    nwin = (hi - base) // W + 1
    sub_a = lax.broadcasted_iota(jnp.int32, (W, B), 0)
    rel0 = aidx_row - base

    def wloop(k, _):
        oh = (sub_a == (rel0 - k * W)).astype(_BF)
        contrib = (jnp.dot(oh, xa_hi, preferred_element_type=_F32)
                   + jnp.dot(oh, xa_lo, preferred_element_type=_F32))
        aacc_ref[pl.ds(base + k * W, W), :] += contrib
        return 0

    lax.fori_loop(0, nwin, wloop, 0)


def _gather_h(lo_ref, hi_ref, aacc_ref, eacc_ref, dist_ref, aidx_ref,
              eidx_ref, w1_ref, i):
    aidx_row = aidx_ref[0]
    eidx_row = eidx_ref[0]

    ecnt = jnp.maximum(eacc_ref[:, 0:1], 1.0)
    pe_tab = jnp.maximum(eacc_ref[:, 8:FS] / ecnt, 0.0)
    pt_hi, pt_lo = _split_hi_lo(pe_tab)
    sub_e = lax.broadcasted_iota(jnp.int32, (TE, B), 0)
    ohe = (sub_e == eidx_row).astype(_BF)
    pe_t = (lax.dot_general(pt_hi, ohe, _C00, preferred_element_type=_F32)
            + lax.dot_general(pt_lo, ohe, _C00, preferred_element_type=_F32))

    lo = lo_ref[i]
    hi = hi_ref[i]
    base = (lo // 8) * 8
    nwin = (hi - base) // W + 1
    sub_a = lax.broadcasted_iota(jnp.int32, (W, B), 0)
    rel0 = aidx_row - base

    def wloop(k, pa_t):
        win = aacc_ref[pl.ds(base + k * W, W), :]
        cnt = jnp.maximum(win[:, N_AE:N_AE + 1], 1.0)
        ptab = jnp.maximum(win[:, :N_AE] / cnt, 0.0)
        at_hi, at_lo = _split_hi_lo(ptab)
        oh = (sub_a == (rel0 - k * W)).astype(_BF)
        return (pa_t
                + lax.dot_general(at_hi, oh, _C00, preferred_element_type=_F32)
                + lax.dot_general(at_lo, oh, _C00, preferred_element_type=_F32))

    pa_t = lax.fori_loop(0, nwin, wloop, jnp.zeros((N_AE, B), _F32))

    pa = pa_t.T
    pe = pe_t.T
    return (jnp.dot(dist_ref[...], w1_ref[0:N_DE, :], preferred_element_type=_F32)
            + jnp.dot(pa, w1_ref[N_DE:N_DE + N_AE, :], preferred_element_type=_F32)
            + jnp.dot(pe, w1_ref[N_DE + N_AE:, :], preferred_element_type=_F32))


def _k1b_body(lo_ref, hi_ref, aacc_ref, eacc_ref, dist_ref, aidx_ref, eidx_ref,
              w1_ref, stats_ref):
    i = pl.program_id(0)
    hb = _gather_h(lo_ref, hi_ref, aacc_ref, eacc_ref, dist_ref, aidx_ref,
                   eidx_ref, w1_ref, i)

    @pl.when(i == 0)
    def _():
        stats_ref[...] = jnp.zeros((8, 128), _F32)

    both = jnp.concatenate([hb, hb * hb], axis=1)
    ones8 = jnp.ones((8, B), _F32)
    stats_ref[...] += jnp.dot(ones8, both, preferred_element_type=_F32)


def _k2_body(lo_ref, hi_ref, aacc_ref, eacc_ref, dist_ref, aidx_ref, eidx_ref,
             w1_ref, x_ref, stats_ref, gamma_ref, beta_ref, out_ref):
    i = pl.program_id(0)
    hb = _gather_h(lo_ref, hi_ref, aacc_ref, eacc_ref, dist_ref, aidx_ref,
                   eidx_ref, w1_ref, i)

    inv_n = 1.0 / N_ROWS
    mu = stats_ref[0:1, 0:64] * inv_n
    ex2 = stats_ref[0:1, 64:128] * inv_n
    var = ex2 - mu * mu
    inv = lax.rsqrt(var + 1e-5)
    scale = gamma_ref[...] * inv
    shift = beta_ref[...] - mu * scale
    out_ref[...] = jnp.maximum(hb * scale + shift + x_ref[...], 0.0)


@jax.jit
def kernel(x, dist_feat, atom_idx, ele_idx, W1, b1, gamma, beta):
    del b1
    aidx = atom_idx.astype(jnp.int32)
    eidx = ele_idx.astype(jnp.int32)
    lo = aidx[::B]
    hi = aidx[B - 1::B]
    aidx3 = aidx.reshape(NB, 1, B)
    eidx3 = eidx.reshape(NB, 1, B)

    grid_a = pltpu.PrefetchScalarGridSpec(
        num_scalar_prefetch=2,
        grid=(NB,),
        in_specs=[
            pl.BlockSpec((B, 64), lambda i, lo, hi: (i, 0)),
            pl.BlockSpec((1, 1, B), lambda i, lo, hi: (i, 0, 0)),
            pl.BlockSpec((1, 1, B), lambda i, lo, hi: (i, 0, 0)),
        ],
        out_specs=[
            pl.BlockSpec((TR, FS), lambda i, lo, hi: (0, 0)),
            pl.BlockSpec((TE, FS), lambda i, lo, hi: (0, 0)),
        ],
    )
    aacc, eacc = pl.pallas_call(
        _k1a_body,
        grid_spec=grid_a,
        out_shape=[
            jax.ShapeDtypeStruct((TR, FS), _F32),
            jax.ShapeDtypeStruct((TE, FS), _F32),
        ],
        compiler_params=pltpu.CompilerParams(
            dimension_semantics=("arbitrary",),
        ),
    )(lo, hi, x, aidx3, eidx3)

    grid_b = pltpu.PrefetchScalarGridSpec(
        num_scalar_prefetch=2,
        grid=(NB,),
        in_specs=[
            pl.BlockSpec((TR, FS), lambda i, lo, hi: (0, 0)),
            pl.BlockSpec((TE, FS), lambda i, lo, hi: (0, 0)),
            pl.BlockSpec((B, N_DE), lambda i, lo, hi: (i, 0)),
            pl.BlockSpec((1, 1, B), lambda i, lo, hi: (i, 0, 0)),
            pl.BlockSpec((1, 1, B), lambda i, lo, hi: (i, 0, 0)),
            pl.BlockSpec((80, 64), lambda i, lo, hi: (0, 0)),
        ],
        out_specs=[
            pl.BlockSpec((8, 128), lambda i, lo, hi: (0, 0)),
        ],
    )
    (stats,) = pl.pallas_call(
        _k1b_body,
        grid_spec=grid_b,
        out_shape=[
            jax.ShapeDtypeStruct((8, 128), _F32),
        ],
        compiler_params=pltpu.CompilerParams(
            dimension_semantics=("arbitrary",),
        ),
    )(lo, hi, aacc, eacc, dist_feat, aidx3, eidx3, W1)

    grid_c = pltpu.PrefetchScalarGridSpec(
        num_scalar_prefetch=2,
        grid=(NB,),
        in_specs=[
            pl.BlockSpec((TR, FS), lambda i, lo, hi: (0, 0)),
            pl.BlockSpec((TE, FS), lambda i, lo, hi: (0, 0)),
            pl.BlockSpec((B, N_DE), lambda i, lo, hi: (i, 0)),
            pl.BlockSpec((1, 1, B), lambda i, lo, hi: (i, 0, 0)),
            pl.BlockSpec((1, 1, B), lambda i, lo, hi: (i, 0, 0)),
            pl.BlockSpec((80, 64), lambda i, lo, hi: (0, 0)),
            pl.BlockSpec((B, 64), lambda i, lo, hi: (i, 0)),
            pl.BlockSpec((8, 128), lambda i, lo, hi: (0, 0)),
            pl.BlockSpec((1, 64), lambda i, lo, hi: (0, 0)),
            pl.BlockSpec((1, 64), lambda i, lo, hi: (0, 0)),
        ],
        out_specs=[
            pl.BlockSpec((B, 64), lambda i, lo, hi: (i, 0)),
        ],
    )
    (out,) = pl.pallas_call(
        _k2_body,
        grid_spec=grid_c,
        out_shape=[
            jax.ShapeDtypeStruct((N_ROWS, 64), jnp.float32),
        ],
        compiler_params=pltpu.CompilerParams(
            dimension_semantics=("arbitrary",),
        ),
    )(lo, hi, aacc, eacc, dist_feat, aidx3, eidx3, W1, x, stats,
      gamma.reshape(1, 64), beta.reshape(1, 64))
    return out


# speedup vs baseline: 1.0468x; 1.0468x over previous
import jax
import jax.numpy as jnp
from jax import lax
from jax.experimental import pallas as pl
from jax.experimental.pallas import tpu as pltpu

N_ROWS = 800000
N_AE = 32
N_DE = 16
N_SEG_ATOM = 50000

B = 1280
NB = N_ROWS // B
W = 128
TR = 50432
TE = 128
FS = 40

_C00 = (((0,), (0,)), ((), ()))
_BF = jnp.bfloat16
_F32 = jnp.float32


def _split_hi_lo(v):
    hi = v.astype(_BF)
    lo = (v - hi.astype(_F32)).astype(_BF)
    return hi, lo


def _k1a_body(lo_ref, hi_ref, x_ref, aidx_ref, eidx_ref, aacc_ref, eacc_ref):
    i = pl.program_id(0)

    @pl.when(i == 0)
    def _():
        aacc_ref[...] = jnp.zeros((TR, FS), _F32)
        eacc_ref[...] = jnp.zeros((TE, FS), _F32)

    aidx_row = aidx_ref[0]
    eidx_row = eidx_ref[0]

    lane40 = lax.broadcasted_iota(jnp.int32, (B, FS), 1)
    x40a = jnp.where(lane40 < N_AE, x_ref[:, 0:FS], 1.0)
    x40e = jnp.where(lane40 >= 8, x_ref[:, 24:64], 1.0)
    sub_e = lax.broadcasted_iota(jnp.int32, (TE, B), 0)
    ohe = (sub_e == eidx_row).astype(_F32)
    eacc_ref[...] += jnp.dot(ohe, x40e, preferred_element_type=_F32)

    lo = lo_ref[i]
    hi = hi_ref[i]
    base = (lo // 8) * 8
    nwin = (hi - base) // W + 1
    sub_a = lax.broadcasted_iota(jnp.int32, (W, B), 0)
    rel0 = aidx_row - base

    def wloop(k, _):
        oh = (sub_a == (rel0 - k * W)).astype(_F32)
        contrib = jnp.dot(oh, x40a, preferred_element_type=_F32)
        aacc_ref[pl.ds(base + k * W, W), :] += contrib
        return 0

    lax.fori_loop(0, nwin, wloop, 0)


def _gather_h(lo_ref, hi_ref, aacc_ref, eacc_ref, dist_ref, aidx_ref,
              eidx_ref, w1_ref, i):
    aidx_row = aidx_ref[0]
    eidx_row = eidx_ref[0]

    ecnt = jnp.maximum(eacc_ref[:, 0:1], 1.0)
    pe_tab = jnp.maximum(eacc_ref[:, 8:FS] / ecnt, 0.0)
    sub_e = lax.broadcasted_iota(jnp.int32, (TE, B), 0)
    ohe = (sub_e == eidx_row).astype(_F32)
    pe_t = lax.dot_general(pe_tab, ohe, _C00, preferred_element_type=_F32)

    lo = lo_ref[i]
    hi = hi_ref[i]
    base = (lo // 8) * 8
    nwin = (hi - base) // W + 1
    sub_a = lax.broadcasted_iota(jnp.int32, (W, B), 0)
    rel0 = aidx_row - base

    def wloop(k, pa_t):
        win = aacc_ref[pl.ds(base + k * W, W), :]
        cnt = jnp.maximum(win[:, N_AE:N_AE + 1], 1.0)
        ptab = jnp.maximum(win[:, :N_AE] / cnt, 0.0)
        oh = (sub_a == (rel0 - k * W)).astype(_F32)
        return pa_t + lax.dot_general(ptab, oh, _C00, preferred_element_type=_F32)

    pa_t = lax.fori_loop(0, nwin, wloop, jnp.zeros((N_AE, B), _F32))

    pa = pa_t.T
    pe = pe_t.T
    return (jnp.dot(dist_ref[...], w1_ref[0:N_DE, :], preferred_element_type=_F32)
            + jnp.dot(pa, w1_ref[N_DE:N_DE + N_AE, :], preferred_element_type=_F32)
            + jnp.dot(pe, w1_ref[N_DE + N_AE:, :], preferred_element_type=_F32))


def _k1b_body(lo_ref, hi_ref, aacc_ref, eacc_ref, dist_ref, aidx_ref, eidx_ref,
              w1_ref, stats_ref):
    i = pl.program_id(0)
    hb = _gather_h(lo_ref, hi_ref, aacc_ref, eacc_ref, dist_ref, aidx_ref,
                   eidx_ref, w1_ref, i)

    @pl.when(i == 0)
    def _():
        stats_ref[...] = jnp.zeros((8, 128), _F32)

    both = jnp.concatenate([hb, hb * hb], axis=1)
    ones8 = jnp.ones((8, B), _F32)
    stats_ref[...] += jnp.dot(ones8, both, preferred_element_type=_F32)


def _k2_body(lo_ref, hi_ref, aacc_ref, eacc_ref, dist_ref, aidx_ref, eidx_ref,
             w1_ref, x_ref, stats_ref, gamma_ref, beta_ref, out_ref):
    i = pl.program_id(0)
    hb = _gather_h(lo_ref, hi_ref, aacc_ref, eacc_ref, dist_ref, aidx_ref,
                   eidx_ref, w1_ref, i)

    inv_n = 1.0 / N_ROWS
    mu = stats_ref[0:1, 0:64] * inv_n
    ex2 = stats_ref[0:1, 64:128] * inv_n
    var = ex2 - mu * mu
    inv = lax.rsqrt(var + 1e-5)
    scale = gamma_ref[...] * inv
    shift = beta_ref[...] - mu * scale
    out_ref[...] = jnp.maximum(hb * scale + shift + x_ref[...], 0.0)


@jax.jit
def kernel(x, dist_feat, atom_idx, ele_idx, W1, b1, gamma, beta):
    del b1
    aidx = atom_idx.astype(jnp.int32)
    eidx = ele_idx.astype(jnp.int32)
    lo = aidx[::B]
    hi = aidx[B - 1::B]
    aidx3 = aidx.reshape(NB, 1, B)
    eidx3 = eidx.reshape(NB, 1, B)

    grid_a = pltpu.PrefetchScalarGridSpec(
        num_scalar_prefetch=2,
        grid=(NB,),
        in_specs=[
            pl.BlockSpec((B, 64), lambda i, lo, hi: (i, 0)),
            pl.BlockSpec((1, 1, B), lambda i, lo, hi: (i, 0, 0)),
            pl.BlockSpec((1, 1, B), lambda i, lo, hi: (i, 0, 0)),
        ],
        out_specs=[
            pl.BlockSpec((TR, FS), lambda i, lo, hi: (0, 0)),
            pl.BlockSpec((TE, FS), lambda i, lo, hi: (0, 0)),
        ],
    )
    aacc, eacc = pl.pallas_call(
        _k1a_body,
        grid_spec=grid_a,
        out_shape=[
            jax.ShapeDtypeStruct((TR, FS), _F32),
            jax.ShapeDtypeStruct((TE, FS), _F32),
        ],
        compiler_params=pltpu.CompilerParams(
            dimension_semantics=("arbitrary",),
        ),
    )(lo, hi, x, aidx3, eidx3)

    grid_b = pltpu.PrefetchScalarGridSpec(
        num_scalar_prefetch=2,
        grid=(NB,),
        in_specs=[
            pl.BlockSpec((TR, FS), lambda i, lo, hi: (0, 0)),
            pl.BlockSpec((TE, FS), lambda i, lo, hi: (0, 0)),
            pl.BlockSpec((B, N_DE), lambda i, lo, hi: (i, 0)),
            pl.BlockSpec((1, 1, B), lambda i, lo, hi: (i, 0, 0)),
            pl.BlockSpec((1, 1, B), lambda i, lo, hi: (i, 0, 0)),
            pl.BlockSpec((80, 64), lambda i, lo, hi: (0, 0)),
        ],
        out_specs=[
            pl.BlockSpec((8, 128), lambda i, lo, hi: (0, 0)),
        ],
    )
    (stats,) = pl.pallas_call(
        _k1b_body,
        grid_spec=grid_b,
        out_shape=[
            jax.ShapeDtypeStruct((8, 128), _F32),
        ],
        compiler_params=pltpu.CompilerParams(
            dimension_semantics=("arbitrary",),
        ),
    )(lo, hi, aacc, eacc, dist_feat, aidx3, eidx3, W1)

    grid_c = pltpu.PrefetchScalarGridSpec(
        num_scalar_prefetch=2,
        grid=(NB,),
        in_specs=[
            pl.BlockSpec((TR, FS), lambda i, lo, hi: (0, 0)),
            pl.BlockSpec((TE, FS), lambda i, lo, hi: (0, 0)),
            pl.BlockSpec((B, N_DE), lambda i, lo, hi: (i, 0)),
            pl.BlockSpec((1, 1, B), lambda i, lo, hi: (i, 0, 0)),
            pl.BlockSpec((1, 1, B), lambda i, lo, hi: (i, 0, 0)),
            pl.BlockSpec((80, 64), lambda i, lo, hi: (0, 0)),
            pl.BlockSpec((B, 64), lambda i, lo, hi: (i, 0)),
            pl.BlockSpec((8, 128), lambda i, lo, hi: (0, 0)),
            pl.BlockSpec((1, 64), lambda i, lo, hi: (0, 0)),
            pl.BlockSpec((1, 64), lambda i, lo, hi: (0, 0)),
        ],
        out_specs=[
            pl.BlockSpec((B, 64), lambda i, lo, hi: (i, 0)),
        ],
    )
    (out,) = pl.pallas_call(
        _k2_body,
        grid_spec=grid_c,
        out_shape=[
            jax.ShapeDtypeStruct((N_ROWS, 64), jnp.float32),
        ],
        compiler_params=pltpu.CompilerParams(
            dimension_semantics=("arbitrary",),
        ),
    )(lo, hi, aacc, eacc, dist_feat, aidx3, eidx3, W1, x, stats,
      gamma.reshape(1, 64), beta.reshape(1, 64))
    return out


# speedup vs baseline: 1.3893x; 1.3272x over previous
import jax
import jax.numpy as jnp
from jax import lax
from jax.experimental import pallas as pl
from jax.experimental.pallas import tpu as pltpu

N_ROWS = 800000
N_AE = 32
N_DE = 16
N_SEG_ATOM = 50000

B = 1280
NB = N_ROWS // B
W = 128
TR = 50432
TE = 128
FS = 40
FD = 56
KC = 8
RC = TR // KC

_C00 = (((0,), (0,)), ((), ()))
_C11 = (((1,), (1,)), ((), ()))
_F32 = jnp.float32


def _k1a_body(lo_ref, hi_ref, x_ref, dist_ref, aidx_ref, eidx_ref,
              aacc_ref, eacc_ref, c_ref, dd_ref):
    i = pl.program_id(0)

    @pl.when(i == 0)
    def _():
        aacc_ref[...] = jnp.zeros((TR, FD), _F32)
        eacc_ref[...] = jnp.zeros((TE, FD), _F32)
        c_ref[...] = jnp.zeros((TR, TE), _F32)
        dd_ref[...] = jnp.zeros((24, 16), _F32)

    aidx_row = aidx_ref[0]
    eidx_row = eidx_ref[0]

    dist = dist_ref[...]
    ones8 = jnp.ones((B, 8), _F32)
    x56a = jnp.concatenate([x_ref[:, 0:N_AE], ones8, dist], axis=1)
    x56e = jnp.concatenate([ones8, x_ref[:, N_AE:2 * N_AE], dist], axis=1)

    sub_e = lax.broadcasted_iota(jnp.int32, (TE, B), 0)
    ohe = (sub_e == eidx_row).astype(_F32)
    eacc_ref[...] += jnp.dot(ohe, x56e, preferred_element_type=_F32)

    dext = jnp.concatenate([dist, ones8], axis=1)
    dd_ref[...] += lax.dot_general(dext, dist, _C00,
                                   preferred_element_type=_F32)

    lo = lo_ref[i]
    hi = hi_ref[i]
    base = (lo // 8) * 8
    nwin = (hi - base) // W + 1
    sub_a = lax.broadcasted_iota(jnp.int32, (W, B), 0)
    rel0 = aidx_row - base

    def wloop(k, _):
        oh = (sub_a == (rel0 - k * W)).astype(_F32)
        aacc_ref[pl.ds(base + k * W, W), :] += jnp.dot(
            oh, x56a, preferred_element_type=_F32)
        c_ref[pl.ds(base + k * W, W), :] += lax.dot_general(
            oh, ohe, _C11, preferred_element_type=_F32)
        return 0

    lax.fori_loop(0, nwin, wloop, 0)


def _pooled(tab, cnt_col, val_cols):
    n = jnp.maximum(tab[:, cnt_col:cnt_col + 1], 1.0)
    p = jnp.maximum(tab[:, val_cols:val_cols + N_AE], 0.0) / n
    return n, p


def _k1b_body(aacc_ref, c_ref, eacc_ref, dd_ref, w1_ref, gamma_ref, beta_ref,
              sl_ref, acc_ref):
    i = pl.program_id(0)

    @pl.when(i == 0)
    def _():
        acc_ref[...] = jnp.zeros((88, 32), _F32)

    tab = aacc_ref[...]
    na, pa = _pooled(tab, N_AE, 0)
    paw = pa * na
    onesc = jnp.ones((RC, 1), _F32)

    eacc = eacc_ref[...]
    ne, pe = _pooled(eacc, 0, 8)

    cpe = jnp.dot(c_ref[...], pe, preferred_element_type=_F32)

    acc_ref[0:32, :] += lax.dot_general(pa, paw, _C00,
                                        preferred_element_type=_F32)
    acc_ref[32:64, :] += lax.dot_general(pa, cpe, _C00,
                                         preferred_element_type=_F32)
    acc_ref[64:80, :] += lax.dot_general(tab[:, FS:FD], pa, _C00,
                                         preferred_element_type=_F32)
    acc_ref[80:81, :] += lax.dot_general(onesc, paw, _C00,
                                         preferred_element_type=_F32)

    @pl.when(i == KC - 1)
    def _():
        qa = acc_ref[0:32, :]
        gae = acc_ref[32:64, :]
        gda = acc_ref[64:80, :]
        sa = acc_ref[80:81, :]

        pew = pe * ne
        onese = jnp.ones((TE, 1), _F32)
        qe = lax.dot_general(pe, pew, _C00, preferred_element_type=_F32)
        gde = lax.dot_general(eacc[:, FS:FD], pe, _C00,
                              preferred_element_type=_F32)
        se = lax.dot_general(onese, pew, _C00, preferred_element_type=_F32)

        gdd = dd_ref[0:16, :]
        sd = dd_ref[16:17, :]

        w1d = w1_ref[0:N_DE, :]
        w1a = w1_ref[N_DE:N_DE + N_AE, :]
        w1e = w1_ref[N_DE + N_AE:, :]

        mu = (jnp.dot(sd, w1d, preferred_element_type=_F32)
              + jnp.dot(sa, w1a, preferred_element_type=_F32)
              + jnp.dot(se, w1e, preferred_element_type=_F32)) * (1.0 / N_ROWS)

        def dsum(a, m, b):
            return jnp.sum(a * jnp.dot(m, b, preferred_element_type=_F32),
                           axis=0, keepdims=True)

        hh = (dsum(w1d, gdd, w1d)
              + 2.0 * dsum(w1d, gda, w1a)
              + 2.0 * dsum(w1d, gde, w1e)
              + dsum(w1a, qa, w1a)
              + 2.0 * dsum(w1a, gae, w1e)
              + dsum(w1e, qe, w1e))

        var = hh * (1.0 / N_ROWS) - mu * mu
        scale = gamma_ref[...] * lax.rsqrt(var + 1e-5)
        shift = beta_ref[...] - mu * scale
        sl_ref[...] = jnp.concatenate(
            [scale, shift, jnp.zeros((6, 64), _F32)], axis=0)


def _k2_body(lo_ref, hi_ref, aacc_ref, eacc_ref, dist_ref, aidx_ref, eidx_ref,
             w1_ref, x_ref, sl_ref, out_ref):
    i = pl.program_id(0)
    aidx_row = aidx_ref[0]
    eidx_row = eidx_ref[0]

    ecnt = jnp.maximum(eacc_ref[:, 0:1], 1.0)
    pe_tab = jnp.maximum(eacc_ref[:, 8:FS], 0.0) / ecnt
    sub_e = lax.broadcasted_iota(jnp.int32, (TE, B), 0)
    ohe = (sub_e == eidx_row).astype(_F32)
    pe_t = lax.dot_general(pe_tab, ohe, _C00, preferred_element_type=_F32)

    lo = lo_ref[i]
    hi = hi_ref[i]
    base = (lo // 8) * 8
    nwin = (hi - base) // W + 1
    sub_a = lax.broadcasted_iota(jnp.int32, (W, B), 0)
    rel0 = aidx_row - base

    def wloop(k, pa_t):
        win = aacc_ref[pl.ds(base + k * W, W), :]
        cnt = jnp.maximum(win[:, N_AE:N_AE + 1], 1.0)
        ptab = jnp.maximum(win[:, :N_AE], 0.0) / cnt
        oh = (sub_a == (rel0 - k * W)).astype(_F32)
        return pa_t + lax.dot_general(ptab, oh, _C00,
                                      preferred_element_type=_F32)

    pa_t = lax.fori_loop(0, nwin, wloop, jnp.zeros((N_AE, B), _F32))

    pa = pa_t.T
    pe = pe_t.T
    hb = (jnp.dot(dist_ref[...], w1_ref[0:N_DE, :], preferred_element_type=_F32)
          + jnp.dot(pa, w1_ref[N_DE:N_DE + N_AE, :], preferred_element_type=_F32)
          + jnp.dot(pe, w1_ref[N_DE + N_AE:, :], preferred_element_type=_F32))

    scale = sl_ref[0:1, :]
    shift = sl_ref[1:2, :]
    out_ref[...] = jnp.maximum(hb * scale + shift + x_ref[...], 0.0)


@jax.jit
def kernel(x, dist_feat, atom_idx, ele_idx, W1, b1, gamma, beta):
    del b1
    aidx = atom_idx.astype(jnp.int32)
    eidx = ele_idx.astype(jnp.int32)
    lo = aidx[::B]
    hi = aidx[B - 1::B]
    aidx3 = aidx.reshape(NB, 1, B)
    eidx3 = eidx.reshape(NB, 1, B)

    grid_a = pltpu.PrefetchScalarGridSpec(
        num_scalar_prefetch=2,
        grid=(NB,),
        in_specs=[
            pl.BlockSpec((B, 64), lambda i, lo, hi: (i, 0)),
            pl.BlockSpec((B, N_DE), lambda i, lo, hi: (i, 0)),
            pl.BlockSpec((1, 1, B), lambda i, lo, hi: (i, 0, 0)),
            pl.BlockSpec((1, 1, B), lambda i, lo, hi: (i, 0, 0)),
        ],
        out_specs=[
            pl.BlockSpec((TR, FD), lambda i, lo, hi: (0, 0)),
            pl.BlockSpec((TE, FD), lambda i, lo, hi: (0, 0)),
            pl.BlockSpec((TR, TE), lambda i, lo, hi: (0, 0)),
            pl.BlockSpec((24, 16), lambda i, lo, hi: (0, 0)),
        ],
    )
    aacc, eacc, ctab, dd = pl.pallas_call(
        _k1a_body,
        grid_spec=grid_a,
        out_shape=[
            jax.ShapeDtypeStruct((TR, FD), _F32),
            jax.ShapeDtypeStruct((TE, FD), _F32),
            jax.ShapeDtypeStruct((TR, TE), _F32),
            jax.ShapeDtypeStruct((24, 16), _F32),
        ],
        compiler_params=pltpu.CompilerParams(
            dimension_semantics=("arbitrary",),
        ),
    )(lo, hi, x, dist_feat, aidx3, eidx3)

    (sl,) = pl.pallas_call(
        _k1b_body,
        grid=(KC,),
        in_specs=[
            pl.BlockSpec((RC, FD), lambda i: (i, 0)),
            pl.BlockSpec((RC, TE), lambda i: (i, 0)),
            pl.BlockSpec((TE, FD), lambda i: (0, 0)),
            pl.BlockSpec((24, 16), lambda i: (0, 0)),
            pl.BlockSpec((80, 64), lambda i: (0, 0)),
            pl.BlockSpec((1, 64), lambda i: (0, 0)),
            pl.BlockSpec((1, 64), lambda i: (0, 0)),
        ],
        out_specs=[
            pl.BlockSpec((8, 64), lambda i: (0, 0)),
        ],
        out_shape=[
            jax.ShapeDtypeStruct((8, 64), _F32),
        ],
        scratch_shapes=[pltpu.VMEM((88, 32), _F32)],
        compiler_params=pltpu.CompilerParams(
            dimension_semantics=("arbitrary",),
        ),
    )(aacc, ctab, eacc, dd, W1, gamma.reshape(1, 64), beta.reshape(1, 64))

    grid_c = pltpu.PrefetchScalarGridSpec(
        num_scalar_prefetch=2,
        grid=(NB,),
        in_specs=[
            pl.BlockSpec((TR, FD), lambda i, lo, hi: (0, 0)),
            pl.BlockSpec((TE, FD), lambda i, lo, hi: (0, 0)),
            pl.BlockSpec((B, N_DE), lambda i, lo, hi: (i, 0)),
            pl.BlockSpec((1, 1, B), lambda i, lo, hi: (i, 0, 0)),
            pl.BlockSpec((1, 1, B), lambda i, lo, hi: (i, 0, 0)),
            pl.BlockSpec((80, 64), lambda i, lo, hi: (0, 0)),
            pl.BlockSpec((B, 64), lambda i, lo, hi: (i, 0)),
            pl.BlockSpec((8, 64), lambda i, lo, hi: (0, 0)),
        ],
        out_specs=[
            pl.BlockSpec((B, 64), lambda i, lo, hi: (i, 0)),
        ],
    )
    (out,) = pl.pallas_call(
        _k2_body,
        grid_spec=grid_c,
        out_shape=[
            jax.ShapeDtypeStruct((N_ROWS, 64), jnp.float32),
        ],
        compiler_params=pltpu.CompilerParams(
            dimension_semantics=("arbitrary",),
        ),
    )(lo, hi, aacc, eacc, dist_feat, aidx3, eidx3, W1, x, sl)
    return out
